# Initial kernel scaffold; baseline (speedup 1.0000x reference)
import jax
import jax.numpy as jnp
from jax.experimental import pallas as pl


def kernel(hm, ct, rg, of):
    raise NotImplementedError("write your pallas kernel here")


# speedup vs baseline: 1.5809x; 1.5809x over previous
import jax
import jax.numpy as jnp
from jax.experimental import pallas as pl
from jax.experimental.pallas import tpu as pltpu

_B = 128
_J = 17
_H0 = 48
_W0 = 48
_HT = 96
_WT = 96
_HIGH = jax.lax.Precision.HIGHEST


def _fiota(shape, dim):
    return jax.lax.broadcasted_iota(jnp.int32, shape, dim).astype(jnp.float32)


def _body(hm_ref, ct_ref, rg_ref, of_ref, out_ref):
    f32 = jnp.float32
    i32 = jnp.int32

    hm0 = hm_ref[0]
    ct0 = ct_ref[0]
    a18 = jnp.concatenate([ct0, hm0], axis=0)

    o_idx = _fiota((_H0, _HT), 1)
    s_idx = _fiota((_H0, _HT), 0)
    s_pos = (o_idx + 0.5) * 0.5 - 0.5
    w_tri = jnp.maximum(0.0, 1.0 - jnp.abs(s_idx - s_pos))
    w_up = w_tri / jnp.sum(w_tri, axis=0, keepdims=True)

    a_x = jnp.dot(a18.reshape(18 * _H0, _W0), w_up,
                  preferred_element_type=f32, precision=_HIGH)
    a_x = jnp.swapaxes(a_x.reshape(18, _H0, _WT), 1, 2)
    rt = jnp.dot(a_x.reshape(18 * _WT, _H0), w_up,
                 preferred_element_type=f32, precision=_HIGH)
    rt = rt.reshape(18, _WT, _HT)
    ct_t = rt[0]
    hm_t = rt[1:]

    lin_t = (jax.lax.broadcasted_iota(i32, (_WT, _HT), 1) * _WT
             + jax.lax.broadcasted_iota(i32, (_WT, _HT), 0))

    m_ct = jnp.max(jnp.max(ct_t, axis=1, keepdims=True), axis=0, keepdims=True)
    ids = jnp.min(jnp.min(jnp.where(ct_t == m_ct, lin_t, _HT * _WT),
                          axis=1, keepdims=True), axis=0, keepdims=True)
    cy = ids // _WT
    cx = ids % _WT

    sy = (cy.astype(f32) + 0.5) * 0.5 - 0.5
    sx = (cx.astype(f32) + 0.5) * 0.5 - 0.5
    y_i = _fiota((_H0, 1), 0)
    w_y = jnp.maximum(0.0, 1.0 - jnp.abs(y_i - sy))
    w_y = (w_y / jnp.sum(w_y, axis=0, keepdims=True)).reshape(1, 1, _H0, 1)
    x_i = _fiota((1, _W0), 1)
    w_x = jnp.maximum(0.0, 1.0 - jnp.abs(x_i - sx))
    w_x = (w_x / jnp.sum(w_x, axis=1, keepdims=True)).reshape(1, 1, _W0)

    rg0 = rg_ref[0]
    rg_v = jnp.sum(jnp.sum(rg0 * w_y, axis=2) * w_x, axis=2)
    reg_x = jnp.clip(cx.astype(f32) + rg_v[:, 0:1] + 0.5, 0.0, _WT - 1.0)
    reg_y = jnp.clip(cy.astype(f32) + rg_v[:, 1:2] + 0.5, 0.0, _HT - 1.0)

    xq = _fiota((1, _WT, _HT), 1)
    yq = _fiota((1, _WT, _HT), 2)
    d2 = ((xq - reg_x.reshape(_J, 1, 1)) ** 2
          + (yq - reg_y.reshape(_J, 1, 1)) ** 2)
    tmp = hm_t / jnp.sqrt(d2 + 1e-9) / 1.8
    m2 = jnp.max(jnp.max(tmp, axis=2, keepdims=True), axis=1, keepdims=True)
    lin3 = lin_t.reshape(1, _WT, _HT)
    ids2 = jnp.min(jnp.min(jnp.where(tmp == m2, lin3, _HT * _WT),
                           axis=2, keepdims=True), axis=1, keepdims=True)
    jy = ids2 // _WT
    jx = ids2 % _WT
    score = jnp.sum(jnp.sum(jnp.where(lin3 == ids2, hm_t, 0.0),
                            axis=2, keepdims=True), axis=1, keepdims=True)

    sy2 = ((jy.astype(f32) + 0.5) * 0.5 - 0.5).reshape(_J, 1, 1, 1)
    sx2 = ((jx.astype(f32) + 0.5) * 0.5 - 0.5).reshape(_J, 1, 1)
    y_i4 = _fiota((1, 1, _H0, 1), 2)
    w_y2 = jnp.maximum(0.0, 1.0 - jnp.abs(y_i4 - sy2))
    w_y2 = w_y2 / jnp.sum(w_y2, axis=2, keepdims=True)
    x_i3 = _fiota((1, 1, _W0), 2)
    w_x2 = jnp.maximum(0.0, 1.0 - jnp.abs(x_i3 - sx2))
    w_x2 = w_x2 / jnp.sum(w_x2, axis=2, keepdims=True)

    of0 = of_ref[0]
    of_v = jnp.sum(jnp.sum(of0 * w_y2, axis=2) * w_x2, axis=2)

    x_norm = (jx.reshape(_J, 1).astype(f32) + of_v[:, 0:1]) / float(_WT)
    y_norm = (jy.reshape(_J, 1).astype(f32) + of_v[:, 1:2]) / float(_HT)
    out = jnp.concatenate([x_norm, y_norm, score.reshape(_J, 1)], axis=1)
    out_ref[0] = out


def kernel(hm, ct, rg, of):
    rg5 = rg.reshape(_B, _J, 2, _H0, _W0)
    of5 = of.reshape(_B, _J, 2, _H0, _W0)
    out = pl.pallas_call(
        _body,
        grid=(_B,),
        in_specs=[
            pl.BlockSpec((1, _J, _H0, _W0), lambda b: (b, 0, 0, 0)),
            pl.BlockSpec((1, 1, _H0, _W0), lambda b: (b, 0, 0, 0)),
            pl.BlockSpec((1, _J, 2, _H0, _W0), lambda b: (b, 0, 0, 0, 0)),
            pl.BlockSpec((1, _J, 2, _H0, _W0), lambda b: (b, 0, 0, 0, 0)),
        ],
        out_specs=pl.BlockSpec((1, _J, 3), lambda b: (b, 0, 0)),
        out_shape=jax.ShapeDtypeStruct((_B, _J, 3), jnp.float32),
        compiler_params=pltpu.CompilerParams(
            dimension_semantics=("arbitrary",),
        ),
    )(hm, ct, rg5, of5)
    return out.reshape(_B, 3 * _J)
